# Initial kernel scaffold; baseline (speedup 1.0000x reference)
import jax
import jax.numpy as jnp
from jax.experimental import pallas as pl


def kernel(image_embedding, bboxes):
    raise NotImplementedError("write your pallas kernel here")


# speedup vs baseline: 2.6729x; 2.6729x over previous
import functools

import jax
import jax.numpy as jnp
from jax import lax
from jax.experimental import pallas as pl
from jax.experimental.pallas import tpu as pltpu
from jax.experimental.pallas import tpu_sc as plsc

B, C, H, W = 4, 96, 48, 48
N = 1000
PH, PW = 7, 7
SCALE = 0.125
CH_HALF = C // 2
CPV = CH_HALF // 16
FEAT_WORDS = H * W * CH_HALF
NBOX = N // 4
OUT_WORDS = CH_HALF * PH * PW

_GATHER_DNUMS = lax.GatherDimensionNumbers(
    offset_dims=(), collapsed_slice_dims=(0,), start_index_map=(0,))


def _lane(vec, p):
    idx = jnp.full((16, 1), p, jnp.int32)
    return lax.gather(vec, idx, _GATHER_DNUMS, (1,),
                      mode=lax.GatherScatterMode.PROMISE_IN_BOUNDS)


def _scalar(vec, p, iota):
    zero = jnp.zeros((16,), vec.dtype)
    return jnp.sum(jnp.where(iota == p, vec, zero))


def _sc_body(feat_hbm, bb_hbm, out_hbm, feat_v, bb_v, ay_v, wy_v, ax_v,
             wx_v, obuf, sem):
    wid = lax.axis_index("s") * 2 + lax.axis_index("c")
    b = wid >> 3
    ch = (wid >> 2) & 1
    n0 = (wid & 3) * NBOX

    pltpu.sync_copy(feat_hbm.at[pl.ds((b * 2 + ch) * FEAT_WORDS, FEAT_WORDS)],
                    feat_v)
    pltpu.sync_copy(bb_hbm.at[pl.ds(b * (N * 4) + n0 * 4, NBOX * 4)], bb_v)

    iota = lax.iota(jnp.int32, 16)
    half = iota & 1
    qf = ((iota >> 1).astype(jnp.float32)
          + half.astype(jnp.float32) * 0.5 + 0.25)
    a_pos = iota * 2
    gy_a = iota >> 2
    gx_b = iota & 3
    ch_off = [iota + 16 * cv for cv in range(CPV)]
    obin = [(iota + 16 * cv) * (PH * PW) for cv in range(CPV)]
    fzero = jnp.zeros((16,), jnp.float32)

    def axis_setup(lo, span, size, stride, idx_ref, w_ref):
        s = lo + span * qf
        valid = (s >= -1.0) & (s <= float(size))
        sc = jnp.clip(s, 0.0, float(size - 1))
        low_i = sc.astype(jnp.int32)
        frac = sc - low_i.astype(jnp.float32)
        high_i = jnp.minimum(low_i + 1, size - 1)
        vhalf = jnp.where(valid, jnp.full((16,), 0.5, jnp.float32), fzero)
        plsc.store_scatter(idx_ref, [a_pos], low_i * stride)
        plsc.store_scatter(idx_ref, [a_pos + 1], high_i * stride)
        plsc.store_scatter(w_ref, [a_pos], (1.0 - frac) * vhalf)
        plsc.store_scatter(w_ref, [a_pos + 1], frac * vhalf)

    def per_box(i, _):
        bbv = plsc.load_gather(bb_v, [i * 4 + gx_b])
        x1 = _scalar(bbv, 0, iota) * SCALE
        y1 = _scalar(bbv, 1, iota) * SCALE
        x2 = _scalar(bbv, 2, iota) * SCALE
        y2 = _scalar(bbv, 3, iota) * SCALE
        bin_w = jnp.maximum(x2 - x1, 1.0) * (1.0 / PW)
        bin_h = jnp.maximum(y2 - y1, 1.0) * (1.0 / PH)
        axis_setup(y1, bin_h, H, W * CH_HALF, ay_v, wy_v)
        axis_setup(x1, bin_w, W, CH_HALF, ax_v, wx_v)

        bufoff = (i & 1) * OUT_WORDS

        oco = ((b * N + n0 + i) * 2 + ch) * OUT_WORDS

        @pl.when(i >= 2)
        def _wait_prev():
            pltpu.make_async_copy(
                obuf.at[pl.ds(bufoff, OUT_WORDS)],
                out_hbm.at[pl.ds(oco, OUT_WORDS)], sem).wait()

        def per_bin(t, _):
            py4 = (t // PW) * 4
            px4 = (t % PW) * 4
            gy = plsc.load_gather(ay_v, [py4 + gy_a])
            wyv = plsc.load_gather(wy_v, [py4 + gy_a])
            gx = plsc.load_gather(ax_v, [px4 + gx_b])
            wxv = plsc.load_gather(wx_v, [px4 + gx_b])
            idx16 = gy + gx
            w16 = wyv * wxv
            acc = [fzero for _ in range(CPV)]
            for p in range(16):
                ip = _lane(idx16, p)
                wp = _lane(w16, p)
                for cv in range(CPV):
                    g = plsc.load_gather(feat_v, [ip + ch_off[cv]])
                    acc[cv] = acc[cv] + g * wp
            for cv in range(CPV):
                plsc.store_scatter(obuf, [obin[cv] + (t + bufoff)], acc[cv])
            return _

        lax.fori_loop(0, PH * PW, per_bin, None)

        pltpu.make_async_copy(
            obuf.at[pl.ds(bufoff, OUT_WORDS)],
            out_hbm.at[pl.ds(oco, OUT_WORDS)], sem).start()
        return _

    lax.fori_loop(0, NBOX, per_box, None)
    for j in range(2):
        pltpu.make_async_copy(
            obuf.at[pl.ds((j % 2) * OUT_WORDS, OUT_WORDS)],
            out_hbm.at[pl.ds((b * N + n0) * 2 * OUT_WORDS, OUT_WORDS)],
            sem).wait()


@jax.jit
def _roi_align_sc(feat, bb):
    mesh = plsc.VectorSubcoreMesh(core_axis_name="c", subcore_axis_name="s",
                                  num_cores=2, num_subcores=16)
    f = pl.kernel(
        _sc_body,
        out_type=jax.ShapeDtypeStruct((B * N * 2 * OUT_WORDS,), jnp.float32),
        mesh=mesh,
        compiler_params=pltpu.CompilerParams(needs_layout_passes=False),
        scratch_types=[
            pltpu.VMEM((FEAT_WORDS,), jnp.float32),
            pltpu.VMEM((NBOX * 4,), jnp.float32),
            pltpu.VMEM((32,), jnp.int32),
            pltpu.VMEM((32,), jnp.float32),
            pltpu.VMEM((32,), jnp.int32),
            pltpu.VMEM((32,), jnp.float32),
            pltpu.VMEM((2 * OUT_WORDS,), jnp.float32),
            pltpu.SemaphoreType.DMA,
        ],
    )
    return f(feat, bb)


def kernel(image_embedding, bboxes):
    feat = jnp.transpose(image_embedding, (0, 2, 3, 1))
    feat = feat.reshape(B, H, W, 2, CH_HALF).transpose(0, 3, 1, 2, 4)
    feat = feat.reshape(B * 2 * FEAT_WORDS)
    bb = bboxes.reshape(B * N * 4)
    out = _roi_align_sc(feat, bb)
    return out.reshape(B, N, C, PH, PW)
